# Initial kernel scaffold; baseline (speedup 1.0000x reference)
import jax
import jax.numpy as jnp
from jax.experimental import pallas as pl


def kernel(node_features, edge_features, edge_index, params):
    raise NotImplementedError("write your pallas kernel here")


# speedup vs baseline: 2.0418x; 2.0418x over previous
import functools

import jax
import jax.numpy as jnp
from jax import lax
from jax.experimental import pallas as pl
from jax.experimental.pallas import tpu as pltpu
from jax.experimental.pallas import tpu_sc as plsc

N = 5000
E = 160000
H = 64
NPAD = 5120
CW = 128
CHUNK = 128
NCHUNK = E // CHUNK
NW = 32
ROWS_PER_TILE = NPAD // 16
_P = lax.Precision.HIGHEST


def _dot(a, b):
    return jnp.dot(a, b)


def _dot_hi(a, b):
    return jnp.dot(a, b, precision=_P)


def _rb(x):
    return x.astype(jnp.bfloat16).astype(jnp.float32)


def _nodes_body(nf, wn1, bn1, wn2, bn2, a0, h_out, hw_out):
    hh = jnp.maximum(_dot(nf[...], wn1[...]) + bn1[...], 0.0)
    h = _dot(hh, wn2[...]) + bn2[...]
    h_out[...] = h
    hw_out[...] = jnp.concatenate(
        [_dot(h, a0[...]), jnp.zeros((N, CW - H), jnp.float32)], axis=1)


def _edges_body(ef, we1, be1, we2, be2, bb0, c0, bb1, c1, ew0_out, ew1_out):
    eh = jnp.maximum(_dot(ef[...], we1[...]) + be1[...], 0.0)
    e = _dot(eh, we2[...]) + be2[...]
    ew0_out[...] = _dot(e, bb0[...]) + c0[...]
    ew1_out[...] = _dot(e, bb1[...]) + c1[...]


def _update_body(part, h_in, w2, b2, v1, c1, v2, c2, pnext, h_out, hw_out):
    p = part[...]
    s = p[0, :N, :H] + p[1, :N, :H]
    deg = p[0, :N, H:H + 1] + p[1, :N, H:H + 1]
    agg = _dot_hi(s, _rb(w2[...])) + deg * b2[...]
    h = h_in[...]
    t = agg + h
    hn = _dot(jnp.maximum(_dot(t, v1[...]) + c1[...], 0.0), v2[...]) + c2[...] + h
    h_out[...] = hn
    hw_out[...] = jnp.concatenate(
        [_dot(hn, pnext[...]), jnp.zeros((N, CW - H), jnp.float32)], axis=1)


def _final_body(part, h_in, w2, b2, v1, c1, v2, c2, o1, bo1, o2, bo2, wk, bk,
                f_out, g_out):
    p = part[...]
    s = p[0, :N, :H] + p[1, :N, :H]
    deg = p[0, :N, H:H + 1] + p[1, :N, H:H + 1]
    agg = _dot_hi(s, _rb(w2[...])) + deg * b2[...]
    h = h_in[...]
    t = agg + h
    h2 = _dot(jnp.maximum(_dot(t, v1[...]) + c1[...], 0.0), v2[...]) + c2[...] + h
    g = _dot(jnp.maximum(_dot(h2, o1[...]) + bo1[...], 0.0), o2[...]) + bo2[...]
    g_out[...] = g
    f_out[...] = _dot(g, wk[...]) + bk[...]


def _head_body(f, g, out):
    out[...] = lax.dot_general(f[...], g[...],
                               dimension_numbers=(((1,), (1,)), ((), ())))


_sc_mesh = plsc.VectorSubcoreMesh(core_axis_name="c", subcore_axis_name="s")


@functools.partial(
    pl.kernel,
    mesh=_sc_mesh,
    out_type=jax.ShapeDtypeStruct((2, NPAD, CW), jnp.float32),
    scratch_types=[
        pltpu.VMEM((CHUNK,), jnp.int32),
        pltpu.VMEM((CHUNK,), jnp.int32),
        pltpu.VMEM((CHUNK, CW), jnp.float32),
        pltpu.VMEM((CHUNK, H), jnp.float32),
        pltpu.VMEM((CHUNK, CW), jnp.float32),
        pltpu.VMEM_SHARED((NPAD, CW), jnp.float32),
    ],
)
def _sc_layer(hw_hbm, ew_hbm, src_hbm, dst_hbm, zeros_hbm, out_hbm,
              src_v, dst_v, gath_v, ew_v, row_v, acc_sh):
    c = lax.axis_index("c")
    s = lax.axis_index("s")
    wid = s * 2 + c
    tslice = pl.ds(s * ROWS_PER_TILE, ROWS_PER_TILE)
    pltpu.sync_copy(zeros_hbm.at[tslice], acc_sh.at[tslice])
    def _init_row(r, carry):
        row_v[r, pl.ds(H, 16)] = jnp.ones((16,), jnp.float32)
        for k in range(H // 16 + 1, CW // 16):
            row_v[r, pl.ds(k * 16, 16)] = jnp.zeros((16,), jnp.float32)
        return carry
    lax.fori_loop(0, CHUNK, _init_row, 0)
    plsc.subcore_barrier()

    nch = jnp.where(wid < NCHUNK - (NCHUNK // NW) * NW, NCHUNK // NW + 1,
                    NCHUNK // NW)

    def _chunk(i, carry):
        base = (wid + i * NW) * CHUNK
        pltpu.sync_copy(src_hbm.at[pl.ds(base, CHUNK)], src_v)
        pltpu.sync_copy(dst_hbm.at[pl.ds(base, CHUNK)], dst_v)
        pltpu.sync_copy(ew_hbm.at[pl.ds(base, CHUNK)], ew_v)
        pltpu.sync_copy(hw_hbm.at[src_v], gath_v)

        def _row(r, carry2):
            for k in range(H // 16):
                sl = pl.ds(k * 16, 16)
                v = jnp.maximum(gath_v[r, sl] + ew_v[r, sl], 0.0)
                u = lax.bitcast_convert_type(v, jnp.int32)
                u = (u + 0x7FFF + ((u >> 16) & 1)) & (-65536)
                row_v[r, sl] = lax.bitcast_convert_type(u, jnp.float32)
            return carry2
        lax.fori_loop(0, CHUNK, _row, 0)
        pltpu.sync_copy(row_v, acc_sh.at[dst_v], add=True)
        return carry
    lax.fori_loop(0, nch, _chunk, 0)

    plsc.subcore_barrier()
    pltpu.sync_copy(acc_sh.at[tslice], out_hbm.at[c, tslice])


def _row(b):
    return b.reshape(1, -1)


def kernel(node_features, edge_features, edge_index, params):
    src = edge_index[0]
    dst = edge_index[1]
    wn1, bn1, wn2, bn2 = params["node_enc"]
    we1, be1, we2, be2 = params["edge_enc"]
    o1, bo1, o2, bo2 = params["mlp_out"]

    a_l, bb_l, c_l, w2_l, b2_l = [], [], [], [], []
    for l in range(2):
        w1, b1, w2, b2 = params["conv_mlp1"][l]
        a_l.append(w1[:H])
        bb_l.append(w1[H:])
        c_l.append(_row(b1))
        w2_l.append(w2)
        b2_l.append(_row(b2))

    _h_hw = [jax.ShapeDtypeStruct((N, H), jnp.float32),
             jax.ShapeDtypeStruct((N, CW), jnp.float32)]
    h0, hw0 = pl.pallas_call(_nodes_body, out_shape=_h_hw)(
        node_features, wn1, _row(bn1), wn2, _row(bn2), a_l[0])

    be_blk = 8000
    grid = (E // be_blk,)
    full = lambda shape: pl.BlockSpec(shape, lambda i: (0, 0))
    ew0, ew1 = pl.pallas_call(
        _edges_body,
        grid=grid,
        in_specs=[
            pl.BlockSpec((be_blk, 16), lambda i: (i, 0)),
            full((16, H)), full((1, H)),
            full((H, H)), full((1, H)),
            full((H, H)), full((1, H)),
            full((H, H)), full((1, H)),
        ],
        out_specs=[pl.BlockSpec((be_blk, H), lambda i: (i, 0))] * 2,
        out_shape=[jax.ShapeDtypeStruct((E, H), jnp.float32)] * 2,
    )(edge_features, we1, _row(be1), we2, _row(be2),
      bb_l[0], c_l[0], bb_l[1], c_l[1])

    zeros = jnp.zeros((NPAD, CW), jnp.float32)

    part0 = _sc_layer(hw0, ew0, src, dst, zeros)
    h1, hw1 = pl.pallas_call(
        _update_body,
        out_shape=_h_hw,
    )(part0, h0, w2_l[0], b2_l[0],
      params["conv_mlp2"][0][0], _row(params["conv_mlp2"][0][1]),
      params["conv_mlp2"][0][2], _row(params["conv_mlp2"][0][3]), a_l[1])

    part1 = _sc_layer(hw1, ew1, src, dst, zeros)
    wk, bk = params["enc"][0]
    f, g = pl.pallas_call(
        _final_body,
        out_shape=[jax.ShapeDtypeStruct((N, H), jnp.float32)] * 2,
    )(part1, h1, w2_l[1], b2_l[1],
      params["conv_mlp2"][1][0], _row(params["conv_mlp2"][1][1]),
      params["conv_mlp2"][1][2], _row(params["conv_mlp2"][1][3]),
      o1, _row(bo1), o2, _row(bo2), wk, _row(bk))

    fp = jnp.zeros((NPAD, H), jnp.float32).at[:N].set(f)
    gp = jnp.zeros((NPAD, H), jnp.float32).at[:N].set(g)
    bm = 640
    nb = NPAD // bm
    out = pl.pallas_call(
        _head_body,
        grid=(nb, nb),
        in_specs=[
            pl.BlockSpec((bm, H), lambda i, j: (i, 0)),
            pl.BlockSpec((bm, H), lambda i, j: (j, 0)),
        ],
        out_specs=pl.BlockSpec((bm, bm), lambda i, j: (i, j)),
        out_shape=jax.ShapeDtypeStruct((NPAD, NPAD), jnp.float32),
    )(fp, gp)
    return out[:N, :N].reshape(N, N, 1)


# speedup vs baseline: 2.2599x; 1.1068x over previous
import functools

import jax
import jax.numpy as jnp
import numpy as np
from jax import lax
from jax.experimental import pallas as pl
from jax.experimental.pallas import tpu as pltpu
from jax.experimental.pallas import tpu_sc as plsc

N = 5000
E = 160000
H = 64
NPAD = 5120
CW = 128
CHUNK = 128
NCHUNK = E // CHUNK
NW = 32
ROWS_PER_TILE = NPAD // 16
_P = lax.Precision.HIGHEST


def _dot(a, b):
    return jnp.dot(a, b)


def _dot_hi(a, b):
    return jnp.dot(a, b, precision=_P)


def _rb(x):
    return x.astype(jnp.bfloat16).astype(jnp.float32)


def _nodes_body(nf, wn1, bn1, wn2, bn2, a0, h_out, hw_out):
    hh = jnp.maximum(_dot(nf[...], wn1[...]) + bn1[...], 0.0)
    h = _dot(hh, wn2[...]) + bn2[...]
    h_out[...] = h
    hw_out[...] = jnp.concatenate(
        [_dot(h, a0[...]), jnp.zeros((N, CW - H), jnp.float32)], axis=1)


def _edges_body(ef, we1, be1, we2, be2, bb0, c0, bb1, c1, ew_out):
    eh = jnp.maximum(_dot(ef[...], we1[...]) + be1[...], 0.0)
    e = _dot(eh, we2[...]) + be2[...]
    ew_out[...] = jnp.concatenate(
        [_dot(e, bb0[...]) + c0[...], _dot(e, bb1[...]) + c1[...]], axis=1)


def _update_body(part, h_in, w2, b2, v1, c1, v2, c2, pnext, h_out, hw_out):
    p = part[...]
    s = p[0, :N, :H] + p[1, :N, :H]
    deg = p[0, :N, H:H + 1] + p[1, :N, H:H + 1]
    agg = _dot_hi(s, _rb(w2[...])) + deg * b2[...]
    h = h_in[...]
    t = agg + h
    hn = _dot(jnp.maximum(_dot(t, v1[...]) + c1[...], 0.0), v2[...]) + c2[...] + h
    h_out[...] = hn
    hw_out[...] = jnp.concatenate(
        [_dot(hn, pnext[...]), jnp.zeros((N, CW - H), jnp.float32)], axis=1)


def _final_body(part, h_in, w2, b2, v1, c1, v2, c2, o1, bo1, o2, bo2, wk, bk,
                f_out, g_out):
    p = part[...]
    s = p[0, :N, :H] + p[1, :N, :H]
    deg = p[0, :N, H:H + 1] + p[1, :N, H:H + 1]
    agg = _dot_hi(s, _rb(w2[...])) + deg * b2[...]
    h = h_in[...]
    t = agg + h
    h2 = _dot(jnp.maximum(_dot(t, v1[...]) + c1[...], 0.0), v2[...]) + c2[...] + h
    g = _dot(jnp.maximum(_dot(h2, o1[...]) + bo1[...], 0.0), o2[...]) + bo2[...]
    g_out[...] = g
    f_out[...] = _dot(g, wk[...]) + bk[...]


def _head_body(f, g, out):
    out[...] = lax.dot_general(f[...], g[...],
                               dimension_numbers=(((1,), (1,)), ((), ())))


_sc_mesh = plsc.VectorSubcoreMesh(core_axis_name="c", subcore_axis_name="s")

AW = 128
CHMAX = NCHUNK // NW + 1
NCHPAD = CHMAX * NW


def _make_sc_layer(off):
    @functools.partial(
        pl.kernel,
        mesh=_sc_mesh,
        out_type=jax.ShapeDtypeStruct((2, NPAD, AW), jnp.float32),
        scratch_types=[
            pltpu.VMEM((CHMAX, CHUNK), jnp.int32),
            pltpu.VMEM((CHMAX, CHUNK), jnp.int32),
            pltpu.VMEM((CHUNK, CW), jnp.float32),
            pltpu.VMEM((CHUNK, CW), jnp.float32),
            pltpu.VMEM((CHUNK, AW), jnp.float32),
            pltpu.VMEM_SHARED((NPAD, AW), jnp.float32),
        ],
    )
    def _sc_layer(hw_hbm, ew_hbm, src_hbm, dst_hbm, zeros_hbm, out_hbm,
                  src_t, dst_t, gath_v, ew_v, row_v, acc_sh):
        c = lax.axis_index("c")
        s = lax.axis_index("s")
        wid = s * 2 + c
        nch = jnp.where(wid < NCHUNK - (NCHUNK // NW) * NW,
                        NCHUNK // NW + 1, NCHUNK // NW)
        start = (NCHUNK // NW) * wid + jnp.minimum(wid, 2)
        tslice = pl.ds(s * ROWS_PER_TILE, ROWS_PER_TILE)
        pltpu.sync_copy(src_hbm.at[pl.ds(wid * CHMAX, CHMAX)], src_t)
        pltpu.sync_copy(dst_hbm.at[pl.ds(wid * CHMAX, CHMAX)], dst_t)
        pltpu.sync_copy(zeros_hbm.at[tslice], acc_sh.at[tslice])
        def _init_row(r, carry):
            row_v[r, pl.ds(H, 16)] = jnp.ones((16,), jnp.float32)
            for k in range(H // 16 + 1, AW // 16):
                row_v[r, pl.ds(k * 16, 16)] = jnp.zeros((16,), jnp.float32)
            return carry
        lax.fori_loop(0, CHUNK, _init_row, 0)
        plsc.subcore_barrier()

        def _chunk(i, carry):
            base = (start + i) * CHUNK
            pltpu.sync_copy(ew_hbm.at[pl.ds(base, CHUNK)], ew_v)
            pltpu.sync_copy(hw_hbm.at[src_t.at[i]], gath_v)

            def _row(r, carry2):
                for k in range(H // 16):
                    sl = pl.ds(k * 16, 16)
                    v = jnp.maximum(
                        gath_v[r, sl] + ew_v[r, pl.ds(off + k * 16, 16)], 0.0)
                    u = lax.bitcast_convert_type(v, jnp.int32)
                    u = (u + 0x7FFF + ((u >> 16) & 1)) & (-65536)
                    row_v[r, sl] = lax.bitcast_convert_type(u, jnp.float32)
                return carry2
            lax.fori_loop(0, CHUNK, _row, 0)
            pltpu.sync_copy(row_v, acc_sh.at[dst_t.at[i]], add=True)
            return carry
        lax.fori_loop(0, nch, _chunk, 0)

        plsc.subcore_barrier()
        pltpu.sync_copy(acc_sh.at[tslice], out_hbm.at[c, tslice])

    return _sc_layer


_sc_layer0 = _make_sc_layer(0)
_sc_layer1 = _make_sc_layer(H)


def _row(b):
    return b.reshape(1, -1)


def kernel(node_features, edge_features, edge_index, params):
    src = edge_index[0]
    dst = edge_index[1]
    wn1, bn1, wn2, bn2 = params["node_enc"]
    we1, be1, we2, be2 = params["edge_enc"]
    o1, bo1, o2, bo2 = params["mlp_out"]

    a_l, bb_l, c_l, w2_l, b2_l = [], [], [], [], []
    for l in range(2):
        w1, b1, w2, b2 = params["conv_mlp1"][l]
        a_l.append(w1[:H])
        bb_l.append(w1[H:])
        c_l.append(_row(b1))
        w2_l.append(w2)
        b2_l.append(_row(b2))

    _h_hw = [jax.ShapeDtypeStruct((N, H), jnp.float32),
             jax.ShapeDtypeStruct((N, CW), jnp.float32)]
    h0, hw0 = pl.pallas_call(_nodes_body, out_shape=_h_hw)(
        node_features, wn1, _row(bn1), wn2, _row(bn2), a_l[0])

    be_blk = 8000
    grid = (E // be_blk,)
    full = lambda shape: pl.BlockSpec(shape, lambda i: (0, 0))
    ew = pl.pallas_call(
        _edges_body,
        grid=grid,
        in_specs=[
            pl.BlockSpec((be_blk, 16), lambda i: (i, 0)),
            full((16, H)), full((1, H)),
            full((H, H)), full((1, H)),
            full((H, H)), full((1, H)),
            full((H, H)), full((1, H)),
        ],
        out_specs=pl.BlockSpec((be_blk, 2 * H), lambda i: (i, 0)),
        out_shape=jax.ShapeDtypeStruct((E, 2 * H), jnp.float32),
    )(edge_features, we1, _row(be1), we2, _row(be2),
      bb_l[0], c_l[0], bb_l[1], c_l[1])

    zeros = jnp.zeros((NPAD, AW), jnp.float32)
    rem = NCHUNK - (NCHUNK // NW) * NW
    ridx = np.array([min((NCHUNK // NW) * w + min(w, rem) + j, NCHUNK - 1)
                     for w in range(NW) for j in range(CHMAX)], np.int32)
    src2 = src.reshape(NCHUNK, CHUNK)[ridx]
    dst2 = dst.reshape(NCHUNK, CHUNK)[ridx]

    part0 = _sc_layer0(hw0, ew, src2, dst2, zeros)
    h1, hw1 = pl.pallas_call(
        _update_body,
        out_shape=_h_hw,
    )(part0, h0, w2_l[0], b2_l[0],
      params["conv_mlp2"][0][0], _row(params["conv_mlp2"][0][1]),
      params["conv_mlp2"][0][2], _row(params["conv_mlp2"][0][3]), a_l[1])

    part1 = _sc_layer1(hw1, ew, src2, dst2, zeros)
    wk, bk = params["enc"][0]
    f, g = pl.pallas_call(
        _final_body,
        out_shape=[jax.ShapeDtypeStruct((N, H), jnp.float32)] * 2,
    )(part1, h1, w2_l[1], b2_l[1],
      params["conv_mlp2"][1][0], _row(params["conv_mlp2"][1][1]),
      params["conv_mlp2"][1][2], _row(params["conv_mlp2"][1][3]),
      o1, _row(bo1), o2, _row(bo2), wk, _row(bk))

    fp = jnp.zeros((NPAD, H), jnp.float32).at[:N].set(f)
    gp = jnp.zeros((NPAD, H), jnp.float32).at[:N].set(g)
    bm = 640
    nb = NPAD // bm
    out = pl.pallas_call(
        _head_body,
        grid=(nb, nb),
        in_specs=[
            pl.BlockSpec((bm, H), lambda i, j: (i, 0)),
            pl.BlockSpec((bm, H), lambda i, j: (j, 0)),
        ],
        out_specs=pl.BlockSpec((bm, bm), lambda i, j: (i, j)),
        out_shape=jax.ShapeDtypeStruct((NPAD, NPAD), jnp.float32),
    )(fp, gp)
    return out[:N, :N].reshape(N, N, 1)


# speedup vs baseline: 2.4678x; 1.0920x over previous
import functools

import jax
import jax.numpy as jnp
import numpy as np
from jax import lax
from jax.experimental import pallas as pl
from jax.experimental.pallas import tpu as pltpu
from jax.experimental.pallas import tpu_sc as plsc

N = 5000
E = 160000
H = 64
NPAD = 5120
CW = 128
CHUNK = 128
NCHUNK = E // CHUNK
NW = 32
ROWS_PER_TILE = NPAD // 16
_P = lax.Precision.HIGHEST


def _dot(a, b):
    return jnp.dot(a, b)


def _dot_hi(a, b):
    return jnp.dot(a, b, precision=_P)


def _rb(x):
    return x.astype(jnp.bfloat16).astype(jnp.float32)


def _nodes_body(nf, wn1, bn1, wn2, bn2, a0, h_out, hw_out):
    hh = jnp.maximum(_dot(nf[...], wn1[...]) + bn1[...], 0.0)
    h = _dot(hh, wn2[...]) + bn2[...]
    h_out[...] = h
    hw_out[...] = jnp.concatenate(
        [_dot(h, a0[...]), jnp.zeros((N, CW - H), jnp.float32)], axis=1)


def _edges_body(ef, we1, be1, we2, be2, bb0, c0, bb1, c1, ew_out):
    eh = jnp.maximum(_dot(ef[...], we1[...]) + be1[...], 0.0)
    e = _dot(eh, we2[...]) + be2[...]
    ew_out[...] = jnp.concatenate(
        [_dot(e, bb0[...]) + c0[...], _dot(e, bb1[...]) + c1[...]], axis=1)


def _update_body(part, h_in, w2, b2, v1, c1, v2, c2, pnext, h_out, hw_out):
    p = part[...]
    s = p[0, :N, :H] + p[1, :N, :H]
    deg = p[0, :N, H:H + 1] + p[1, :N, H:H + 1]
    agg = _dot_hi(s, _rb(w2[...])) + deg * b2[...]
    h = h_in[...]
    t = agg + h
    hn = _dot(jnp.maximum(_dot(t, v1[...]) + c1[...], 0.0), v2[...]) + c2[...] + h
    h_out[...] = hn
    hw_out[...] = jnp.concatenate(
        [_dot(hn, pnext[...]), jnp.zeros((N, CW - H), jnp.float32)], axis=1)


def _final_body(part, h_in, w2, b2, v1, c1, v2, c2, o1, bo1, o2, bo2, wk, bk,
                f_out, g_out):
    p = part[...]
    s = p[0, :N, :H] + p[1, :N, :H]
    deg = p[0, :N, H:H + 1] + p[1, :N, H:H + 1]
    agg = _dot_hi(s, _rb(w2[...])) + deg * b2[...]
    h = h_in[...]
    t = agg + h
    h2 = _dot(jnp.maximum(_dot(t, v1[...]) + c1[...], 0.0), v2[...]) + c2[...] + h
    g = _dot(jnp.maximum(_dot(h2, o1[...]) + bo1[...], 0.0), o2[...]) + bo2[...]
    g_out[...] = g
    f_out[...] = _dot(g, wk[...]) + bk[...]


def _head_body(f, g, out):
    out[...] = lax.dot_general(f[...], g[...],
                               dimension_numbers=(((1,), (1,)), ((), ())))


_sc_mesh = plsc.VectorSubcoreMesh(core_axis_name="c", subcore_axis_name="s")

AW = 128
CHMAX = NCHUNK // NW + 1
NCHPAD = CHMAX * NW


def _make_sc_layer(off):
    @functools.partial(
        pl.kernel,
        mesh=_sc_mesh,
        out_type=jax.ShapeDtypeStruct((2, NPAD, AW), jnp.float32),
        scratch_types=[
            pltpu.VMEM((CHMAX, CHUNK), jnp.int32),
            pltpu.VMEM((CHMAX, CHUNK), jnp.int32),
            pltpu.VMEM((CHUNK, CW), jnp.float32),
            pltpu.VMEM((CHUNK, CW), jnp.float32),
            pltpu.VMEM((CHUNK, CW), jnp.float32),
            pltpu.VMEM((CHUNK, AW), jnp.float32),
            pltpu.VMEM_SHARED((NPAD, AW), jnp.float32),
            pltpu.SemaphoreType.DMA,
            pltpu.SemaphoreType.DMA,
        ],
    )
    def _sc_layer(hw_hbm, ew_hbm, src_hbm, dst_hbm, out_hbm,
                  src_t, dst_t, ew_v, g_a, g_b, row_v, acc_sh,
                  sem_a, sem_b):
        c = lax.axis_index("c")
        s = lax.axis_index("s")
        wid = s * 2 + c
        rem = NCHUNK - (NCHUNK // NW) * NW
        nch = jnp.where(wid < rem, NCHUNK // NW + 1, NCHUNK // NW)
        start = (NCHUNK // NW) * wid + jnp.minimum(wid, rem)
        tslice = pl.ds(s * ROWS_PER_TILE, ROWS_PER_TILE)
        pltpu.sync_copy(src_hbm.at[pl.ds(wid * CHMAX, CHMAX)], src_t)
        pltpu.sync_copy(dst_hbm.at[pl.ds(wid * CHMAX, CHMAX)], dst_t)
        def _zero_row(r, carry):
            for k in range(CW // 16):
                ew_v[r, pl.ds(k * 16, 16)] = jnp.zeros((16,), jnp.float32)
            return carry
        lax.fori_loop(0, CHUNK, _zero_row, 0)
        for t in range(ROWS_PER_TILE // CHUNK):
            pltpu.sync_copy(ew_v,
                            acc_sh.at[pl.ds(s * ROWS_PER_TILE + t * CHUNK,
                                            CHUNK)])
        rtail = ROWS_PER_TILE % CHUNK
        if rtail:
            pltpu.sync_copy(
                ew_v.at[pl.ds(0, rtail)],
                acc_sh.at[pl.ds(
                    s * ROWS_PER_TILE + (ROWS_PER_TILE // CHUNK) * CHUNK,
                    rtail)])
        def _init_row(r, carry):
            for k in range(H // 16, AW // 16):
                row_v[r, pl.ds(k * 16, 16)] = (
                    jnp.ones((16,), jnp.float32) if k == H // 16
                    else jnp.zeros((16,), jnp.float32))
            return carry
        lax.fori_loop(0, CHUNK, _init_row, 0)
        plsc.subcore_barrier()

        def _issue(g_v, sem, i):
            pltpu.async_copy(hw_hbm.at[src_t.at[i]], g_v, sem)

        def _drain(g_v, sem, i):
            pltpu.make_async_copy(
                hw_hbm.at[src_t.at[i]], g_v, sem).wait()

        def _compute_scatter(g_v, i):
            base = (start + i) * CHUNK
            pltpu.sync_copy(ew_hbm.at[pl.ds(base, CHUNK)], ew_v)
            def _rowf(r, carry2):
                for k in range(H // 16):
                    sl = pl.ds(k * 16, 16)
                    v = jnp.maximum(
                        g_v[r, sl] + ew_v[r, pl.ds(off + k * 16, 16)], 0.0)
                    u = lax.bitcast_convert_type(v, jnp.int32)
                    u = (u + 0x7FFF + ((u >> 16) & 1)) & (-65536)
                    row_v[r, sl] = lax.bitcast_convert_type(u, jnp.float32)
                return carry2
            lax.fori_loop(0, CHUNK, _rowf, 0)
            pltpu.sync_copy(row_v, acc_sh.at[dst_t.at[i]], add=True)

        _issue(g_a, sem_a, 0)

        def _pair(j, carry):
            i0 = 2 * j
            i1 = 2 * j + 1

            @pl.when(i1 < nch)
            def _():
                _issue(g_b, sem_b, i1)

            @pl.when(i0 < nch)
            def _():
                _drain(g_a, sem_a, i0)
                _compute_scatter(g_a, i0)

            @pl.when(i0 + 2 < nch)
            def _():
                _issue(g_a, sem_a, i0 + 2)

            @pl.when(i1 < nch)
            def _():
                _drain(g_b, sem_b, i1)
                _compute_scatter(g_b, i1)
            return carry
        lax.fori_loop(0, CHMAX // 2, _pair, 0)

        plsc.subcore_barrier()
        pltpu.sync_copy(acc_sh.at[tslice], out_hbm.at[c, tslice])

    return _sc_layer


_sc_layer0 = _make_sc_layer(0)
_sc_layer1 = _make_sc_layer(H)


def _row(b):
    return b.reshape(1, -1)


def kernel(node_features, edge_features, edge_index, params):
    src = edge_index[0]
    dst = edge_index[1]
    wn1, bn1, wn2, bn2 = params["node_enc"]
    we1, be1, we2, be2 = params["edge_enc"]
    o1, bo1, o2, bo2 = params["mlp_out"]

    a_l, bb_l, c_l, w2_l, b2_l = [], [], [], [], []
    for l in range(2):
        w1, b1, w2, b2 = params["conv_mlp1"][l]
        a_l.append(w1[:H])
        bb_l.append(w1[H:])
        c_l.append(_row(b1))
        w2_l.append(w2)
        b2_l.append(_row(b2))

    _h_hw = [jax.ShapeDtypeStruct((N, H), jnp.float32),
             jax.ShapeDtypeStruct((N, CW), jnp.float32)]
    h0, hw0 = pl.pallas_call(_nodes_body, out_shape=_h_hw)(
        node_features, wn1, _row(bn1), wn2, _row(bn2), a_l[0])

    be_blk = 8000
    grid = (E // be_blk,)
    full = lambda shape: pl.BlockSpec(shape, lambda i: (0, 0))
    ew = pl.pallas_call(
        _edges_body,
        grid=grid,
        in_specs=[
            pl.BlockSpec((be_blk, 16), lambda i: (i, 0)),
            full((16, H)), full((1, H)),
            full((H, H)), full((1, H)),
            full((H, H)), full((1, H)),
            full((H, H)), full((1, H)),
        ],
        out_specs=pl.BlockSpec((be_blk, 2 * H), lambda i: (i, 0)),
        out_shape=jax.ShapeDtypeStruct((E, 2 * H), jnp.float32),
    )(edge_features, we1, _row(be1), we2, _row(be2),
      bb_l[0], c_l[0], bb_l[1], c_l[1])

    rem = NCHUNK - (NCHUNK // NW) * NW
    ridx = np.array([min((NCHUNK // NW) * w + min(w, rem) + j, NCHUNK - 1)
                     for w in range(NW) for j in range(CHMAX)], np.int32)
    src2 = src.reshape(NCHUNK, CHUNK)[ridx]
    dst2 = dst.reshape(NCHUNK, CHUNK)[ridx]

    part0 = _sc_layer0(hw0, ew, src2, dst2)
    h1, hw1 = pl.pallas_call(
        _update_body,
        out_shape=_h_hw,
    )(part0, h0, w2_l[0], b2_l[0],
      params["conv_mlp2"][0][0], _row(params["conv_mlp2"][0][1]),
      params["conv_mlp2"][0][2], _row(params["conv_mlp2"][0][3]), a_l[1])

    part1 = _sc_layer1(hw1, ew, src2, dst2)
    wk, bk = params["enc"][0]
    f, g = pl.pallas_call(
        _final_body,
        out_shape=[jax.ShapeDtypeStruct((N, H), jnp.float32)] * 2,
    )(part1, h1, w2_l[1], b2_l[1],
      params["conv_mlp2"][1][0], _row(params["conv_mlp2"][1][1]),
      params["conv_mlp2"][1][2], _row(params["conv_mlp2"][1][3]),
      o1, _row(bo1), o2, _row(bo2), wk, _row(bk))

    fp = jnp.zeros((NPAD, H), jnp.float32).at[:N].set(f)
    gp = jnp.zeros((NPAD, H), jnp.float32).at[:N].set(g)
    bm = 640
    nb = NPAD // bm
    out = pl.pallas_call(
        _head_body,
        grid=(nb, nb),
        in_specs=[
            pl.BlockSpec((bm, H), lambda i, j: (i, 0)),
            pl.BlockSpec((bm, H), lambda i, j: (j, 0)),
        ],
        out_specs=pl.BlockSpec((bm, bm), lambda i, j: (i, j)),
        out_shape=jax.ShapeDtypeStruct((NPAD, NPAD), jnp.float32),
    )(fp, gp)
    return out[:N, :N].reshape(N, N, 1)
